# baseline (speedup 1.0000x reference)
import functools

import jax
import jax.numpy as jnp
from jax import lax
from jax.experimental import pallas as pl
from jax.experimental.pallas import tpu as pltpu
from jax.experimental.pallas import tpu_sc as plsc


_NC = 2
_NS = 16
_NW = _NC * _NS
_LANES = 16
_MAX_TILE = 800
_NBUF = 12


@functools.lru_cache(maxsize=None)
def _make_sc_broadcast(rows: int, d: int):
    assert rows % _NW == 0
    rows_per_w = rows // _NW
    tile = _MAX_TILE
    while rows_per_w % tile or tile % 8:
        tile -= 8
    steps = rows_per_w // tile
    nbuf = min(_NBUF, steps)
    assert d % _LANES == 0 and rows_per_w % 8 == 0

    mesh = plsc.VectorSubcoreMesh(core_axis_name="c", subcore_axis_name="s")

    @functools.partial(
        pl.kernel,
        mesh=mesh,
        out_type=jax.ShapeDtypeStruct((rows, d), jnp.float32),
        scratch_types=[
            pltpu.VMEM((tile, d), jnp.float32),
            pltpu.SemaphoreType.DMA,
        ],
    )
    def sc_broadcast(w_hbm, out_hbm, tile_v, sem):
        wid = lax.axis_index("s") * _NC + lax.axis_index("c")
        base = wid * rows_per_w

        pltpu.sync_copy(w_hbm, tile_v.at[pl.ds(0, 1)])
        vregs = [tile_v[0, pl.ds(_LANES * j, _LANES)] for j in range(d // _LANES)]

        def fill(r, carry):
            for j in range(d // _LANES):
                tile_v[r, pl.ds(_LANES * j, _LANES)] = vregs[j]
            return carry

        lax.fori_loop(1, tile, fill, 0)

        for t in range(nbuf):
            pltpu.async_copy(tile_v, out_hbm.at[pl.ds(base + t * tile, tile)], sem)

        def body(t, carry):
            pltpu.make_async_copy(
                tile_v, out_hbm.at[pl.ds(base, tile)], sem
            ).wait()
            pltpu.async_copy(
                tile_v, out_hbm.at[pl.ds(base + t * tile, tile)], sem
            )
            return carry

        lax.fori_loop(nbuf, steps, body, 0)

        for _ in range(nbuf):
            pltpu.make_async_copy(tile_v, out_hbm.at[pl.ds(base, tile)], sem).wait()

    return sc_broadcast


def kernel(input, weight):
    B, H = input.shape
    _, D = weight.shape
    out = _make_sc_broadcast(B * H, D)(weight)
    return out.reshape(B, H, D)


# speedup vs baseline: 1.1058x; 1.1058x over previous
import functools

import jax
import jax.numpy as jnp
from jax import lax
from jax.experimental import pallas as pl
from jax.experimental.pallas import tpu as pltpu


_BLOCK_ROWS = 8192
_NQ = 4
_NBUF = 2


def _make_tc_ring(rows: int, d: int):
    block = _BLOCK_ROWS
    while rows % (block * _NQ):
        block //= 2
    steps = rows // block

    def body(w_ref, o_ref, *scratch):
        bufs = scratch[:_NQ]
        sems = scratch[_NQ:]
        for q in range(_NQ):
            bufs[q][...] = jnp.broadcast_to(w_ref[...], bufs[q].shape)

        for t in range(_NQ * _NBUF):
            pltpu.make_async_copy(
                bufs[t % _NQ], o_ref.at[pl.ds(t * block, block)], sems[t % _NQ]
            ).start()

        def ring(t, carry):
            q = lax.rem(t, _NQ)

            def fire(qq):
                pltpu.make_async_copy(
                    bufs[qq], o_ref.at[pl.ds(0, block)], sems[qq]
                ).wait()
                pltpu.make_async_copy(
                    bufs[qq], o_ref.at[pl.ds(t * block, block)], sems[qq]
                ).start()

            lax.switch(q, [functools.partial(fire, qq) for qq in range(_NQ)])
            return carry

        lax.fori_loop(_NQ * _NBUF, steps, ring, 0)

        for t in range(_NQ * _NBUF):
            pltpu.make_async_copy(
                bufs[t % _NQ], o_ref.at[pl.ds(0, block)], sems[t % _NQ]
            ).wait()

    return pl.pallas_call(
        body,
        in_specs=[pl.BlockSpec(memory_space=pltpu.MemorySpace.VMEM)],
        out_specs=pl.BlockSpec(memory_space=pl.ANY),
        out_shape=jax.ShapeDtypeStruct((rows, d), jnp.float32),
        scratch_shapes=(
            [pltpu.VMEM((block, d), jnp.float32)] * _NQ
            + [pltpu.SemaphoreType.DMA] * _NQ
        ),
    )


def kernel(input, weight):
    B, H = input.shape
    _, D = weight.shape
    out = _make_tc_ring(B * H, D)(weight)
    return out.reshape(B, H, D)
